# baseline (speedup 1.0000x reference)
import functools

import jax
import jax.numpy as jnp
from jax import lax
from jax.experimental import pallas as pl
from jax.experimental.pallas import tpu as pltpu
from jax.experimental.pallas import tpu_sc as plsc

D = 16
T = 4
NW = 32
CHUNK = 128
NCHUNK = 50

_mesh = plsc.VectorSubcoreMesh(core_axis_name="c", subcore_axis_name="s")


@functools.partial(
    pl.kernel,
    mesh=_mesh,
    compiler_params=pltpu.CompilerParams(use_tc_tiling_on_sc=False),
    out_type=jax.ShapeDtypeStruct((T, NW, NCHUNK, CHUNK, D), jnp.float32),
    scratch_types=[
        pltpu.VMEM((NCHUNK, CHUNK), jnp.int32),
        pltpu.VMEM((CHUNK, D), jnp.float32),
        pltpu.VMEM((T, CHUNK, D), jnp.float32),
        pltpu.SemaphoreType.DMA,
    ],
)
def _lif_embed(table_hbm, idx_hbm, out_hbm, idx_v, rows_v, outs_v, sem):
    wid = lax.axis_index("s") * 2 + lax.axis_index("c")
    pltpu.sync_copy(idx_hbm.at[wid], idx_v)

    def chunk_body(j, carry):
        pltpu.async_copy(table_hbm.at[idx_v.at[j]], rows_v, sem).wait()

        def tok_body(i, c):
            e = rows_v[i]
            m = e
            for t in range(T):
                s = jnp.where(m >= 1.0, 1.0, 0.0)
                outs_v[t, i] = s
                m = (m - s) * 0.5 + e
            return c

        lax.fori_loop(0, CHUNK, tok_body, 0)
        for t in range(T):
            pltpu.sync_copy(outs_v.at[t], out_hbm.at[t, wid, j])
        return carry

    lax.fori_loop(0, NCHUNK, chunk_body, 0)


def kernel(input_ids, embedding_weight):
    B, L = input_ids.shape
    ids = input_ids.astype(jnp.int32).reshape(NW, NCHUNK, CHUNK)
    out = _lif_embed(embedding_weight, ids)
    return out.reshape(T, B, L, D)


# speedup vs baseline: 1.0422x; 1.0422x over previous
import functools

import jax
import jax.numpy as jnp
from jax import lax
from jax.experimental import pallas as pl
from jax.experimental.pallas import tpu as pltpu
from jax.experimental.pallas import tpu_sc as plsc

D = 16
T = 4
NW = 32
CHUNK = 128
NCHUNK = 50

_mesh = plsc.VectorSubcoreMesh(core_axis_name="c", subcore_axis_name="s")


@functools.partial(
    pl.kernel,
    mesh=_mesh,
    compiler_params=pltpu.CompilerParams(use_tc_tiling_on_sc=False),
    out_type=jax.ShapeDtypeStruct((T, NW, NCHUNK, CHUNK, D), jnp.float32),
    scratch_types=[
        pltpu.VMEM((NCHUNK, CHUNK), jnp.int32),
        pltpu.VMEM((2, CHUNK, D), jnp.float32),
        pltpu.VMEM((2, T, CHUNK, D), jnp.float32),
        pltpu.SemaphoreType.DMA,
        pltpu.SemaphoreType.DMA,
    ],
)
def _lif_embed(table_hbm, idx_hbm, out_hbm, idx_v, rows_v, outs_v, gsem, ssem):
    wid = lax.axis_index("s") * 2 + lax.axis_index("c")
    pltpu.sync_copy(idx_hbm.at[wid], idx_v)
    pltpu.async_copy(table_hbm.at[idx_v.at[0]], rows_v.at[0], gsem)

    def chunk_body(j, carry):
        b = lax.rem(j, 2)

        @pl.when(j + 1 < NCHUNK)
        def _():
            pltpu.async_copy(table_hbm.at[idx_v.at[j + 1]], rows_v.at[1 - b], gsem)

        pltpu.make_async_copy(table_hbm.at[idx_v.at[j]], rows_v.at[b], gsem).wait()

        @pl.when(j >= 2)
        def _():
            for t in range(T):
                pltpu.make_async_copy(
                    outs_v.at[b, t], out_hbm.at[t, wid, j - 2], ssem
                ).wait()

        rb = rows_v.at[b]
        ob = outs_v.at[b]

        @plsc.parallel_loop(0, CHUNK, unroll=8)
        def tok(i):
            e = rb[i]
            m = e
            for t in range(T):
                s = jnp.where(m >= 1.0, 1.0, 0.0)
                ob[t, i] = s
                if t < T - 1:
                    m = (m - s) * 0.5 + e

        for t in range(T):
            pltpu.async_copy(ob.at[t], out_hbm.at[t, wid, j], ssem)
        return carry

    lax.fori_loop(0, NCHUNK, chunk_body, 0)

    for jj in (NCHUNK - 2, NCHUNK - 1):
        for t in range(T):
            pltpu.make_async_copy(
                outs_v.at[jj % 2, t], out_hbm.at[t, wid, jj], ssem
            ).wait()


def kernel(input_ids, embedding_weight):
    B, L = input_ids.shape
    ids = input_ids.astype(jnp.int32).reshape(NW, NCHUNK, CHUNK)
    out = _lif_embed(embedding_weight, ids)
    return out.reshape(T, B, L, D)


# speedup vs baseline: 2.1254x; 2.0393x over previous
import functools

import jax
import jax.numpy as jnp
from jax import lax
from jax.experimental import pallas as pl
from jax.experimental.pallas import tpu as pltpu
from jax.experimental.pallas import tpu_sc as plsc

D = 16
T = 4
NW = 32
CHUNK = 128
L = 50

_mesh = plsc.VectorSubcoreMesh(core_axis_name="c", subcore_axis_name="s")


@functools.partial(
    pl.kernel,
    mesh=_mesh,
    compiler_params=pltpu.CompilerParams(
        use_tc_tiling_on_sc=False, needs_layout_passes=False
    ),
    out_type=jax.ShapeDtypeStruct((T, L, 2, NW, 8, 128), jnp.float32),
    scratch_types=[
        pltpu.VMEM((L, CHUNK), jnp.int32),
        pltpu.VMEM((2, CHUNK, D), jnp.float32),
        pltpu.VMEM((2, T, D, CHUNK), jnp.float32),
        pltpu.SemaphoreType.DMA,
        pltpu.SemaphoreType.DMA,
    ],
)
def _lif_embed(table_hbm, ids_hbm, out_hbm, ids_v, rows_v, outs_v, gsem, ssem):
    wid = lax.axis_index("s") * 2 + lax.axis_index("c")
    pltpu.sync_copy(ids_hbm.at[:, pl.ds(wid * CHUNK, CHUNK)], ids_v)
    pltpu.async_copy(table_hbm.at[ids_v.at[0]], rows_v.at[0], gsem)

    lane = jnp.arange(16, dtype=jnp.int32)

    def chunk_body(l, carry):
        b = lax.rem(l, 2)

        @pl.when(l + 1 < L)
        def _():
            pltpu.async_copy(table_hbm.at[ids_v.at[l + 1]], rows_v.at[1 - b], gsem)

        pltpu.make_async_copy(table_hbm.at[ids_v.at[l]], rows_v.at[b], gsem).wait()

        @pl.when(l >= 2)
        def _():
            for t in range(T):
                for dt in range(2):
                    pltpu.make_async_copy(
                        outs_v.at[b, t, pl.ds(dt * 8, 8), :],
                        out_hbm.at[t, l - 2, dt, wid],
                        ssem,
                    ).wait()

        rows = rows_v.at[b]

        @plsc.parallel_loop(0, CHUNK, unroll=4)
        def tok(k):
            d = lax.div(k, 8)
            jj = lax.rem(k, 8)
            e = plsc.load_gather(rows, [jj * 16 + lane, jnp.full((16,), d, jnp.int32)])
            m = e
            for t in range(T):
                s = jnp.where(m >= 1.0, 1.0, 0.0)
                outs_v[b, t, d, pl.ds(jj * 16, 16)] = s
                if t < T - 1:
                    m = (m - s) * 0.5 + e

        for t in range(T):
            for dt in range(2):
                pltpu.async_copy(
                    outs_v.at[b, t, pl.ds(dt * 8, 8), :],
                    out_hbm.at[t, l, dt, wid],
                    ssem,
                )
        return carry

    lax.fori_loop(0, L, chunk_body, 0)

    for ll in (L - 2, L - 1):
        for t in range(T):
            for dt in range(2):
                pltpu.make_async_copy(
                    outs_v.at[ll % 2, t, pl.ds(dt * 8, 8), :],
                    out_hbm.at[t, ll, dt, wid],
                    ssem,
                ).wait()


def kernel(input_ids, embedding_weight):
    B, Lx = input_ids.shape
    ids_t = input_ids.astype(jnp.int32).T
    out6 = _lif_embed(embedding_weight, ids_t)
    return out6.transpose(0, 3, 5, 1, 2, 4).reshape(T, B, Lx, D)


# speedup vs baseline: 3.6714x; 1.7274x over previous
import functools

import jax
import jax.numpy as jnp
from jax import lax
from jax.experimental import pallas as pl
from jax.experimental.pallas import tpu as pltpu
from jax.experimental.pallas import tpu_sc as plsc

D = 16
T = 4
NW = 32
CHUNK = 128
L = 50

_mesh = plsc.VectorSubcoreMesh(core_axis_name="c", subcore_axis_name="s")

NCOL = 7813
NFULL = NCOL - 1


@functools.partial(
    pl.kernel,
    mesh=_mesh,
    compiler_params=pltpu.CompilerParams(
        use_tc_tiling_on_sc=True, needs_layout_passes=False
    ),
    out_type=jax.ShapeDtypeStruct((125000, 128), jnp.float32),
    scratch_types=[
        pltpu.VMEM((2, D, 128), jnp.float32),
        pltpu.VMEM((2, D, 128), jnp.float32),
        pltpu.SemaphoreType.DMA,
        pltpu.SemaphoreType.DMA,
    ],
)
def _detranspose(tt_hbm, tail_hbm, out_hbm, stage_v, trans_v, isem, osem):
    wid = lax.axis_index("s") * 2 + lax.axis_index("c")
    nfull = jnp.where(wid < NFULL - 244 * 32, 245, 244)
    lane = jnp.arange(16, dtype=jnp.int32)

    pltpu.async_copy(tt_hbm.at[:, pl.ds(wid * 128, 128)], stage_v.at[0], isem)

    def col_body(k, carry):
        b = lax.rem(k, 2)
        c = wid + k * 32

        @pl.when(k + 1 < nfull)
        def _():
            pltpu.async_copy(
                tt_hbm.at[:, pl.ds((c + 32) * 128, 128)], stage_v.at[1 - b], isem
            )

        pltpu.make_async_copy(
            tt_hbm.at[:, pl.ds(c * 128, 128)], stage_v.at[b], isem
        ).wait()

        @pl.when(k >= 2)
        def _():
            pltpu.make_async_copy(
                trans_v.at[b], out_hbm.at[pl.ds((c - 64) * 16, 16), :], osem
            ).wait()

        sb = stage_v.at[b]
        tb = trans_v.at[b]

        @plsc.parallel_loop(0, 128, unroll=4)
        def vtx(j):
            col = plsc.load_gather(sb, [lane, jnp.full((16,), j, jnp.int32)])
            tb[lax.div(j, 8), pl.ds(lax.rem(j, 8) * 16, 16)] = col

        pltpu.async_copy(trans_v.at[b], out_hbm.at[pl.ds(c * 16, 16), :], osem)
        return carry

    lax.fori_loop(0, nfull, col_body, 0)

    for dk in (2, 1):
        k = nfull - dk
        b = lax.rem(k, 2)
        c = wid + k * 32
        pltpu.make_async_copy(
            trans_v.at[b], out_hbm.at[pl.ds(c * 16, 16), :], osem
        ).wait()

    @pl.when(wid == (NFULL % 32))
    def _():
        pltpu.sync_copy(tail_hbm, stage_v.at[0, pl.ds(0, 8), :])
        pltpu.sync_copy(
            stage_v.at[0, pl.ds(0, 8), :], out_hbm.at[pl.ds(NFULL * 16, 8), :]
        )


@functools.partial(
    pl.kernel,
    mesh=_mesh,
    compiler_params=pltpu.CompilerParams(
        use_tc_tiling_on_sc=False, needs_layout_passes=False
    ),
    out_type=jax.ShapeDtypeStruct((T, L, 2, NW, 8, 128), jnp.float32),
    scratch_types=[
        pltpu.VMEM((L, CHUNK), jnp.int32),
        pltpu.VMEM((2, CHUNK, D), jnp.float32),
        pltpu.VMEM((2, T, D, CHUNK), jnp.float32),
        pltpu.SemaphoreType.DMA,
        pltpu.SemaphoreType.DMA,
    ],
)
def _lif_embed(table_hbm, ids_hbm, out_hbm, ids_v, rows_v, outs_v, gsem, ssem):
    wid = lax.axis_index("s") * 2 + lax.axis_index("c")
    pltpu.sync_copy(ids_hbm.at[:, pl.ds(wid * CHUNK, CHUNK)], ids_v)
    pltpu.async_copy(table_hbm.at[ids_v.at[0]], rows_v.at[0], gsem)

    lane = jnp.arange(16, dtype=jnp.int32)

    def chunk_body(l, carry):
        b = lax.rem(l, 2)

        @pl.when(l + 1 < L)
        def _():
            pltpu.async_copy(table_hbm.at[ids_v.at[l + 1]], rows_v.at[1 - b], gsem)

        pltpu.make_async_copy(table_hbm.at[ids_v.at[l]], rows_v.at[b], gsem).wait()

        @pl.when(l >= 2)
        def _():
            for t in range(T):
                for dt in range(2):
                    pltpu.make_async_copy(
                        outs_v.at[b, t, pl.ds(dt * 8, 8), :],
                        out_hbm.at[t, l - 2, dt, wid],
                        ssem,
                    ).wait()

        rows = rows_v.at[b]

        @plsc.parallel_loop(0, CHUNK, unroll=4)
        def tok(k):
            d = lax.div(k, 8)
            jj = lax.rem(k, 8)
            e = plsc.load_gather(rows, [jj * 16 + lane, jnp.full((16,), d, jnp.int32)])
            m = e
            for t in range(T):
                s = jnp.where(m >= 1.0, 1.0, 0.0)
                outs_v[b, t, d, pl.ds(jj * 16, 16)] = s
                if t < T - 1:
                    m = (m - s) * 0.5 + e

        for t in range(T):
            for dt in range(2):
                pltpu.async_copy(
                    outs_v.at[b, t, pl.ds(dt * 8, 8), :],
                    out_hbm.at[t, l, dt, wid],
                    ssem,
                )
        return carry

    lax.fori_loop(0, L, chunk_body, 0)

    for ll in (L - 2, L - 1):
        for t in range(T):
            for dt in range(2):
                pltpu.make_async_copy(
                    outs_v.at[ll % 2, t, pl.ds(dt * 8, 8), :],
                    out_hbm.at[t, ll, dt, wid],
                    ssem,
                ).wait()


def kernel(input_ids, embedding_weight):
    B, Lx = input_ids.shape
    ids_t = input_ids.astype(jnp.int32).T
    tail = embedding_weight[NFULL * 128:].reshape(8, 128)
    table_g = _detranspose(embedding_weight.T, tail)
    out6 = _lif_embed(table_g.reshape(1000000, 16), ids_t)
    return out6.transpose(0, 3, 5, 1, 2, 4).reshape(T, B, Lx, D)
